# Initial kernel scaffold; baseline (speedup 1.0000x reference)
import jax
import jax.numpy as jnp
from jax.experimental import pallas as pl


def kernel(data, params):
    raise NotImplementedError("write your pallas kernel here")


# speedup vs baseline: 1.0159x; 1.0159x over previous
import functools

import jax
import jax.numpy as jnp
from jax.experimental import pallas as pl


_TR = 2048


def _mm_stats(x, W, b):
    rows, cin = x.shape
    cout = W.shape[1]
    tr = min(rows, _TR)
    grid = rows // tr

    def kfn(x_ref, w_ref, b_ref, y_ref, s1_ref, s2_ref):
        i = pl.program_id(0)
        y = jnp.dot(x_ref[...], w_ref[...],
                    preferred_element_type=jnp.float32) + b_ref[...]
        y_ref[...] = y

        @pl.when(i == 0)
        def _():
            s1_ref[...] = jnp.zeros_like(s1_ref)
            s2_ref[...] = jnp.zeros_like(s2_ref)

        s1_ref[...] += jnp.sum(y, axis=0, keepdims=True)
        s2_ref[...] += jnp.sum(y * y, axis=0, keepdims=True)

    y, s1, s2 = pl.pallas_call(
        kfn,
        grid=(grid,),
        in_specs=[
            pl.BlockSpec((tr, cin), lambda i: (i, 0)),
            pl.BlockSpec((cin, cout), lambda i: (0, 0)),
            pl.BlockSpec((1, cout), lambda i: (0, 0)),
        ],
        out_specs=[
            pl.BlockSpec((tr, cout), lambda i: (i, 0)),
            pl.BlockSpec((1, cout), lambda i: (0, 0)),
            pl.BlockSpec((1, cout), lambda i: (0, 0)),
        ],
        out_shape=[
            jax.ShapeDtypeStruct((rows, cout), jnp.float32),
            jax.ShapeDtypeStruct((1, cout), jnp.float32),
            jax.ShapeDtypeStruct((1, cout), jnp.float32),
        ],
    )(x, W, b.reshape(1, -1))
    return y, s1, s2


def _aff_relu_mm_stats(x, A, Bv, W, b):
    rows, cin = x.shape
    cout = W.shape[1]
    tr = min(rows, _TR)
    grid = rows // tr

    def kfn(x_ref, a_ref, bv_ref, w_ref, b_ref, y_ref, s1_ref, s2_ref):
        i = pl.program_id(0)
        h = jnp.maximum(x_ref[...] * a_ref[...] + bv_ref[...], 0.0)
        y = jnp.dot(h, w_ref[...], preferred_element_type=jnp.float32) + b_ref[...]
        y_ref[...] = y

        @pl.when(i == 0)
        def _():
            s1_ref[...] = jnp.zeros_like(s1_ref)
            s2_ref[...] = jnp.zeros_like(s2_ref)

        s1_ref[...] += jnp.sum(y, axis=0, keepdims=True)
        s2_ref[...] += jnp.sum(y * y, axis=0, keepdims=True)

    y, s1, s2 = pl.pallas_call(
        kfn,
        grid=(grid,),
        in_specs=[
            pl.BlockSpec((tr, cin), lambda i: (i, 0)),
            pl.BlockSpec((1, cin), lambda i: (0, 0)),
            pl.BlockSpec((1, cin), lambda i: (0, 0)),
            pl.BlockSpec((cin, cout), lambda i: (0, 0)),
            pl.BlockSpec((1, cout), lambda i: (0, 0)),
        ],
        out_specs=[
            pl.BlockSpec((tr, cout), lambda i: (i, 0)),
            pl.BlockSpec((1, cout), lambda i: (0, 0)),
            pl.BlockSpec((1, cout), lambda i: (0, 0)),
        ],
        out_shape=[
            jax.ShapeDtypeStruct((rows, cout), jnp.float32),
            jax.ShapeDtypeStruct((1, cout), jnp.float32),
            jax.ShapeDtypeStruct((1, cout), jnp.float32),
        ],
    )(x, A, Bv, W, b.reshape(1, -1))
    return y, s1, s2


def _aff_relu_mm_maskmax(x, A, Bv, W, b, mask, k):
    rows, cin = x.shape
    cout = W.shape[1]
    q_total = rows // k
    tq = min(q_total, max(1, _TR // k))
    tr = tq * k
    grid = q_total // tq

    def kfn(x_ref, a_ref, bv_ref, w_ref, b_ref, m_ref, o_ref):
        h = jnp.maximum(x_ref[...] * a_ref[...] + bv_ref[...], 0.0)
        y = jnp.dot(h, w_ref[...], preferred_element_type=jnp.float32) + b_ref[...]
        y = y.reshape(tq, k, cout)
        m = m_ref[...].reshape(tq, k, 1)
        y = jnp.where(m > 0, y, -1e30)
        o_ref[...] = jnp.max(y, axis=1)

    out = pl.pallas_call(
        kfn,
        grid=(grid,),
        in_specs=[
            pl.BlockSpec((tr, cin), lambda i: (i, 0)),
            pl.BlockSpec((1, cin), lambda i: (0, 0)),
            pl.BlockSpec((1, cin), lambda i: (0, 0)),
            pl.BlockSpec((cin, cout), lambda i: (0, 0)),
            pl.BlockSpec((1, cout), lambda i: (0, 0)),
            pl.BlockSpec((tq, k), lambda i: (i, 0)),
        ],
        out_specs=pl.BlockSpec((tq, cout), lambda i: (i, 0)),
        out_shape=jax.ShapeDtypeStruct((q_total, cout), jnp.float32),
    )(x, A, Bv, W, b.reshape(1, -1), mask)
    return out


def _aff_relu_max(x, A, Bv, k):
    rows, cin = x.shape
    q_total = rows // k
    tq = min(q_total, max(1, _TR // k))
    tr = tq * k
    grid = q_total // tq

    def kfn(x_ref, a_ref, bv_ref, o_ref):
        h = jnp.maximum(x_ref[...] * a_ref[...] + bv_ref[...], 0.0)
        o_ref[...] = jnp.max(h.reshape(tq, k, cin), axis=1)

    out = pl.pallas_call(
        kfn,
        grid=(grid,),
        in_specs=[
            pl.BlockSpec((tr, cin), lambda i: (i, 0)),
            pl.BlockSpec((1, cin), lambda i: (0, 0)),
            pl.BlockSpec((1, cin), lambda i: (0, 0)),
        ],
        out_specs=pl.BlockSpec((tq, cin), lambda i: (i, 0)),
        out_shape=jax.ShapeDtypeStruct((q_total, cin), jnp.float32),
    )(x, A, Bv)
    return out


def _affine_from_stats(s1, s2, rows, g, be):
    mu = s1 / rows
    var = s2 / rows - mu * mu
    rstd = jax.lax.rsqrt(var + 1e-5)
    A = g.reshape(1, -1) * rstd
    Bv = be.reshape(1, -1) - mu * A
    return A, Bv


def _fps(pos, n_samples):
    dist = jnp.sum((pos - pos[0]) ** 2, axis=-1)
    idx0 = jnp.zeros((n_samples,), jnp.int32)

    def body(i, carry):
        idx, d = carry
        nxt = jnp.argmax(d).astype(jnp.int32)
        idx = idx.at[i].set(nxt)
        d = jnp.minimum(d, jnp.sum((pos - pos[nxt]) ** 2, axis=-1))
        return (idx, d)

    idx, _ = jax.lax.fori_loop(1, n_samples, body, (idx0, dist))
    return idx


def _knn_idx(pos, qpos, k):
    qq = jnp.sum(qpos * qpos, axis=-1)[:, None]
    pp = jnp.sum(pos * pos, axis=-1)[None, :]
    d2 = qq + pp - 2.0 * (qpos @ pos.T)
    neg, idx = jax.lax.top_k(-d2, k)
    return idx, -neg


def _gather_nodes(x, idx):
    return jnp.take_along_axis(x, idx[..., None], axis=1)


def _gather_neighbors(x, nbr):
    b, q, k = nbr.shape
    out = jnp.take_along_axis(x, nbr.reshape(b, q * k)[..., None], axis=1)
    return out.reshape(b, q, k, x.shape[-1])


def _mlp_rows(x, layers, plain_last=True, use_norm=True):
    rows = x.shape[0]
    n = len(layers)
    y, s1, s2 = _mm_stats(x, layers[0]["W"], layers[0]["b"])
    for i in range(1, n):
        L0 = layers[i - 1]
        if use_norm:
            A, Bv = _affine_from_stats(s1, s2, rows, L0["g"], L0["be"])
        else:
            A = jnp.ones((1, y.shape[1]), jnp.float32)
            Bv = jnp.zeros((1, y.shape[1]), jnp.float32)
        y, s1, s2 = _aff_relu_mm_stats(y, A, Bv, layers[i]["W"], layers[i]["b"])
    if not plain_last:
        Ln = layers[-1]
        if use_norm:
            A, Bv = _affine_from_stats(s1, s2, rows, Ln["g"], Ln["be"])
        else:
            A = jnp.ones((1, y.shape[1]), jnp.float32)
            Bv = jnp.zeros((1, y.shape[1]), jnp.float32)
        return y, A, Bv
    return y


def _sa_module(x, pos, ratio, r, layers, k=64):
    b, n, c = x.shape
    n_s = int(n * ratio)
    idx = jax.vmap(_fps, in_axes=(0, None))(pos, n_s)
    qpos = _gather_nodes(pos, idx)
    nbr, d2 = jax.vmap(_knn_idx, in_axes=(0, 0, None))(pos, qpos, k)
    valid = d2 <= r * r
    xj = _gather_neighbors(x, nbr)
    posj = _gather_neighbors(pos, nbr)
    rel = posj - qpos[:, :, None, :]
    feat = jnp.concatenate([xj, rel], axis=-1).reshape(b * n_s * k, c + 3)
    rows = feat.shape[0]

    y, s1, s2 = _mm_stats(feat, layers[0]["W"], layers[0]["b"])
    for i in range(1, len(layers) - 1):
        A, Bv = _affine_from_stats(s1, s2, rows, layers[i - 1]["g"], layers[i - 1]["be"])
        y, s1, s2 = _aff_relu_mm_stats(y, A, Bv, layers[i]["W"], layers[i]["b"])
    A, Bv = _affine_from_stats(s1, s2, rows, layers[-2]["g"], layers[-2]["be"])
    mask = valid.reshape(b * n_s, k).astype(jnp.float32)
    out = _aff_relu_mm_maskmax(y, A, Bv, layers[-1]["W"], layers[-1]["b"], mask, k)
    return out.reshape(b, n_s, -1), qpos


def _transition_down(x, pos, layers, ratio=0.25, k=16):
    b, n, c = x.shape
    n_s = int(n * ratio)
    idx = jax.vmap(_fps, in_axes=(0, None))(pos, n_s)
    qpos = _gather_nodes(pos, idx)
    y, s1, s2 = _mm_stats(x.reshape(b * n, c), layers[0]["W"], layers[0]["b"])
    A, Bv = _affine_from_stats(s1, s2, b * n, layers[0]["g"], layers[0]["be"])
    cout = y.shape[1]
    nbr, _ = jax.vmap(_knn_idx, in_axes=(0, 0, None))(pos, qpos, k)
    hj = _gather_neighbors(y.reshape(b, n, cout), nbr).reshape(b * n_s * k, cout)
    out = _aff_relu_max(hj, A, Bv, k)
    return out.reshape(b, n_s, cout), qpos


def kernel(data, params):
    x = data
    pos = data
    x, pos = _sa_module(x, pos, 0.5, 0.2, params["sa1"], k=64)
    x, pos = _transition_down(x, pos, params["td1"], ratio=0.25, k=16)
    x, pos = _sa_module(x, pos, 0.25, 0.4, params["sa2"], k=64)
    x, pos = _transition_down(x, pos, params["td2"], ratio=0.25, k=16)
    b = data.shape[0]
    feat = jnp.concatenate([x, pos], axis=-1)
    h = _mlp_rows(feat.reshape(-1, feat.shape[-1]), params["sa3"])
    g = jnp.mean(h.reshape(b, -1, h.shape[-1]), axis=1)
    y = _mlp_rows(g, params["head"], plain_last=True, use_norm=False)
    return y


# speedup vs baseline: 1.9178x; 1.8878x over previous
import functools

import jax
import jax.numpy as jnp
from jax.experimental import pallas as pl


_TR = 2048


def _stats_update(i, tr, y, s1_ref, s2_ref):
    mb = jnp.mean(y, axis=0, keepdims=True)
    m2b = jnp.sum((y - mb) ** 2, axis=0, keepdims=True)

    @pl.when(i == 0)
    def _():
        s1_ref[...] = mb
        s2_ref[...] = m2b

    @pl.when(i > 0)
    def _():
        na = (i * tr).astype(jnp.float32)
        tot = na + tr
        delta = mb - s1_ref[...]
        s1_ref[...] += delta * (tr / tot)
        s2_ref[...] += m2b + delta * delta * (na * tr / tot)


def _mm_stats(x, W, b):
    rows, cin = x.shape
    cout = W.shape[1]
    tr = min(rows, _TR)
    grid = rows // tr

    def kfn(x_ref, w_ref, b_ref, y_ref, s1_ref, s2_ref):
        i = pl.program_id(0)
        y = jnp.dot(x_ref[...], w_ref[...],
                    preferred_element_type=jnp.float32) + b_ref[...]
        y_ref[...] = y
        _stats_update(i, tr, y, s1_ref, s2_ref)

    y, s1, s2 = pl.pallas_call(
        kfn,
        grid=(grid,),
        in_specs=[
            pl.BlockSpec((tr, cin), lambda i: (i, 0)),
            pl.BlockSpec((cin, cout), lambda i: (0, 0)),
            pl.BlockSpec((1, cout), lambda i: (0, 0)),
        ],
        out_specs=[
            pl.BlockSpec((tr, cout), lambda i: (i, 0)),
            pl.BlockSpec((1, cout), lambda i: (0, 0)),
            pl.BlockSpec((1, cout), lambda i: (0, 0)),
        ],
        out_shape=[
            jax.ShapeDtypeStruct((rows, cout), jnp.float32),
            jax.ShapeDtypeStruct((1, cout), jnp.float32),
            jax.ShapeDtypeStruct((1, cout), jnp.float32),
        ],
    )(x, W, b.reshape(1, -1))
    return y, s1, s2


def _aff_relu_mm_stats(x, Mu, A, Bv, W, b):
    rows, cin = x.shape
    cout = W.shape[1]
    tr = min(rows, _TR)
    grid = rows // tr

    def kfn(x_ref, mu_ref, a_ref, bv_ref, w_ref, b_ref, y_ref, s1_ref, s2_ref):
        i = pl.program_id(0)
        h = jnp.maximum((x_ref[...] - mu_ref[...]) * a_ref[...] + bv_ref[...], 0.0)
        y = jnp.dot(h, w_ref[...], preferred_element_type=jnp.float32) + b_ref[...]
        y_ref[...] = y
        _stats_update(i, tr, y, s1_ref, s2_ref)

    y, s1, s2 = pl.pallas_call(
        kfn,
        grid=(grid,),
        in_specs=[
            pl.BlockSpec((tr, cin), lambda i: (i, 0)),
            pl.BlockSpec((1, cin), lambda i: (0, 0)),
            pl.BlockSpec((1, cin), lambda i: (0, 0)),
            pl.BlockSpec((1, cin), lambda i: (0, 0)),
            pl.BlockSpec((cin, cout), lambda i: (0, 0)),
            pl.BlockSpec((1, cout), lambda i: (0, 0)),
        ],
        out_specs=[
            pl.BlockSpec((tr, cout), lambda i: (i, 0)),
            pl.BlockSpec((1, cout), lambda i: (0, 0)),
            pl.BlockSpec((1, cout), lambda i: (0, 0)),
        ],
        out_shape=[
            jax.ShapeDtypeStruct((rows, cout), jnp.float32),
            jax.ShapeDtypeStruct((1, cout), jnp.float32),
            jax.ShapeDtypeStruct((1, cout), jnp.float32),
        ],
    )(x, Mu, A, Bv, W, b.reshape(1, -1))
    return y, s1, s2


def _aff_relu_mm_maskmax(x, Mu, A, Bv, W, b, mask, k):
    rows, cin = x.shape
    cout = W.shape[1]
    q_total = rows // k
    tq = min(q_total, max(1, _TR // k))
    tr = tq * k
    grid = q_total // tq

    def kfn(x_ref, mu_ref, a_ref, bv_ref, w_ref, b_ref, m_ref, o_ref):
        h = jnp.maximum((x_ref[...] - mu_ref[...]) * a_ref[...] + bv_ref[...], 0.0)
        y = jnp.dot(h, w_ref[...], preferred_element_type=jnp.float32) + b_ref[...]
        y = y.reshape(tq, k, cout)
        m = m_ref[...].reshape(tq, k, 1)
        y = jnp.where(m > 0, y, -1e30)
        o_ref[...] = jnp.max(y, axis=1)

    out = pl.pallas_call(
        kfn,
        grid=(grid,),
        in_specs=[
            pl.BlockSpec((tr, cin), lambda i: (i, 0)),
            pl.BlockSpec((1, cin), lambda i: (0, 0)),
            pl.BlockSpec((1, cin), lambda i: (0, 0)),
            pl.BlockSpec((1, cin), lambda i: (0, 0)),
            pl.BlockSpec((cin, cout), lambda i: (0, 0)),
            pl.BlockSpec((1, cout), lambda i: (0, 0)),
            pl.BlockSpec((tq, k), lambda i: (i, 0)),
        ],
        out_specs=pl.BlockSpec((tq, cout), lambda i: (i, 0)),
        out_shape=jax.ShapeDtypeStruct((q_total, cout), jnp.float32),
    )(x, Mu, A, Bv, W, b.reshape(1, -1), mask)
    return out


def _aff_relu_max(x, Mu, A, Bv, k):
    rows, cin = x.shape
    q_total = rows // k
    tq = min(q_total, max(1, _TR // k))
    tr = tq * k
    grid = q_total // tq

    def kfn(x_ref, mu_ref, a_ref, bv_ref, o_ref):
        h = jnp.maximum((x_ref[...] - mu_ref[...]) * a_ref[...] + bv_ref[...], 0.0)
        o_ref[...] = jnp.max(h.reshape(tq, k, cin), axis=1)

    out = pl.pallas_call(
        kfn,
        grid=(grid,),
        in_specs=[
            pl.BlockSpec((tr, cin), lambda i: (i, 0)),
            pl.BlockSpec((1, cin), lambda i: (0, 0)),
            pl.BlockSpec((1, cin), lambda i: (0, 0)),
            pl.BlockSpec((1, cin), lambda i: (0, 0)),
        ],
        out_specs=pl.BlockSpec((tq, cin), lambda i: (i, 0)),
        out_shape=jax.ShapeDtypeStruct((q_total, cin), jnp.float32),
    )(x, Mu, A, Bv)
    return out


def _affine_from_stats(s1, s2, rows, g, be):
    mu = s1
    var = s2 / rows
    rstd = jax.lax.rsqrt(var + 1e-5)
    A = g.reshape(1, -1) * rstd
    Bv = be.reshape(1, -1)
    return mu, A, Bv


def _fps_all(pos, n_samples):
    bsz, n, _ = pos.shape
    px = pos[..., 0]
    py = pos[..., 1]
    pz = pos[..., 2]

    def kfn(px_ref, py_ref, pz_ref, o_ref):
        pxv = px_ref[...]
        pyv = py_ref[...]
        pzv = pz_ref[...]
        lane = jax.lax.broadcasted_iota(jnp.int32, (bsz, n), 1)
        olane = jax.lax.broadcasted_iota(jnp.int32, (bsz, n_samples), 1)
        qx = pxv[:, 0:1]
        qy = pyv[:, 0:1]
        qz = pzv[:, 0:1]
        d = (pxv - qx) ** 2 + (pyv - qy) ** 2 + (pzv - qz) ** 2
        o_ref[...] = jnp.zeros((bsz, n_samples), jnp.int32)

        def body(i, d):
            nxt = jnp.argmax(d, axis=1).astype(jnp.int32)[:, None]
            o_ref[...] = jnp.where(olane == i, nxt, o_ref[...])
            m = lane == nxt
            qx = jnp.sum(jnp.where(m, pxv, 0.0), axis=1, keepdims=True)
            qy = jnp.sum(jnp.where(m, pyv, 0.0), axis=1, keepdims=True)
            qz = jnp.sum(jnp.where(m, pzv, 0.0), axis=1, keepdims=True)
            dn = (pxv - qx) ** 2 + (pyv - qy) ** 2 + (pzv - qz) ** 2
            return jnp.minimum(d, dn)

        jax.lax.fori_loop(1, n_samples, body, d)

    return pl.pallas_call(
        kfn,
        out_shape=jax.ShapeDtypeStruct((bsz, n_samples), jnp.int32),
    )(px, py, pz)


def _knn_idx(pos, qpos, k):
    qq = jnp.sum(qpos * qpos, axis=-1)[:, None]
    pp = jnp.sum(pos * pos, axis=-1)[None, :]
    d2 = qq + pp - 2.0 * (qpos @ pos.T)
    neg, idx = jax.lax.top_k(-d2, k)
    return idx, -neg


def _gather_nodes(x, idx):
    return jnp.take_along_axis(x, idx[..., None], axis=1)


def _gather_neighbors(x, nbr):
    b, q, k = nbr.shape
    out = jnp.take_along_axis(x, nbr.reshape(b, q * k)[..., None], axis=1)
    return out.reshape(b, q, k, x.shape[-1])


def _mlp_rows(x, layers, plain_last=True, use_norm=True):
    rows = x.shape[0]
    n = len(layers)
    y, s1, s2 = _mm_stats(x, layers[0]["W"], layers[0]["b"])
    for i in range(1, n):
        L0 = layers[i - 1]
        if use_norm:
            Mu, A, Bv = _affine_from_stats(s1, s2, rows, L0["g"], L0["be"])
        else:
            Mu = jnp.zeros((1, y.shape[1]), jnp.float32)
            A = jnp.ones((1, y.shape[1]), jnp.float32)
            Bv = jnp.zeros((1, y.shape[1]), jnp.float32)
        y, s1, s2 = _aff_relu_mm_stats(y, Mu, A, Bv, layers[i]["W"], layers[i]["b"])
    if not plain_last:
        Ln = layers[-1]
        if use_norm:
            Mu, A, Bv = _affine_from_stats(s1, s2, rows, Ln["g"], Ln["be"])
        else:
            Mu = jnp.zeros((1, y.shape[1]), jnp.float32)
            A = jnp.ones((1, y.shape[1]), jnp.float32)
            Bv = jnp.zeros((1, y.shape[1]), jnp.float32)
        return y, Mu, A, Bv
    return y


def _sa_module(x, pos, ratio, r, layers, k=64):
    b, n, c = x.shape
    n_s = int(n * ratio)
    idx = _fps_all(pos, n_s)
    qpos = _gather_nodes(pos, idx)
    nbr, d2 = jax.vmap(_knn_idx, in_axes=(0, 0, None))(pos, qpos, k)
    valid = d2 <= r * r
    xj = _gather_neighbors(x, nbr)
    posj = _gather_neighbors(pos, nbr)
    rel = posj - qpos[:, :, None, :]
    feat = jnp.concatenate([xj, rel], axis=-1).reshape(b * n_s * k, c + 3)
    rows = feat.shape[0]

    y, s1, s2 = _mm_stats(feat, layers[0]["W"], layers[0]["b"])
    for i in range(1, len(layers) - 1):
        Mu, A, Bv = _affine_from_stats(s1, s2, rows, layers[i - 1]["g"], layers[i - 1]["be"])
        y, s1, s2 = _aff_relu_mm_stats(y, Mu, A, Bv, layers[i]["W"], layers[i]["b"])
    Mu, A, Bv = _affine_from_stats(s1, s2, rows, layers[-2]["g"], layers[-2]["be"])
    mask = valid.reshape(b * n_s, k).astype(jnp.float32)
    out = _aff_relu_mm_maskmax(y, Mu, A, Bv, layers[-1]["W"], layers[-1]["b"], mask, k)
    return out.reshape(b, n_s, -1), qpos


def _transition_down(x, pos, layers, ratio=0.25, k=16):
    b, n, c = x.shape
    n_s = int(n * ratio)
    idx = _fps_all(pos, n_s)
    qpos = _gather_nodes(pos, idx)
    y, s1, s2 = _mm_stats(x.reshape(b * n, c), layers[0]["W"], layers[0]["b"])
    Mu, A, Bv = _affine_from_stats(s1, s2, b * n, layers[0]["g"], layers[0]["be"])
    cout = y.shape[1]
    nbr, _ = jax.vmap(_knn_idx, in_axes=(0, 0, None))(pos, qpos, k)
    hj = _gather_neighbors(y.reshape(b, n, cout), nbr).reshape(b * n_s * k, cout)
    out = _aff_relu_max(hj, Mu, A, Bv, k)
    return out.reshape(b, n_s, cout), qpos


def kernel(data, params):
    x = data
    pos = data
    x, pos = _sa_module(x, pos, 0.5, 0.2, params["sa1"], k=64)
    x, pos = _transition_down(x, pos, params["td1"], ratio=0.25, k=16)
    x, pos = _sa_module(x, pos, 0.25, 0.4, params["sa2"], k=64)
    x, pos = _transition_down(x, pos, params["td2"], ratio=0.25, k=16)
    b = data.shape[0]
    feat = jnp.concatenate([x, pos], axis=-1)
    h = _mlp_rows(feat.reshape(-1, feat.shape[-1]), params["sa3"])
    g = jnp.mean(h.reshape(b, -1, h.shape[-1]), axis=1)
    y = _mlp_rows(g, params["head"], plain_last=True, use_norm=False)
    return y
